# Initial kernel scaffold; baseline (speedup 1.0000x reference)
import jax
import jax.numpy as jnp
from jax.experimental import pallas as pl


def kernel(coordinates, features, weights):
    raise NotImplementedError("write your pallas kernel here")


# speedup vs baseline: 31.9125x; 31.9125x over previous
import functools

import jax
import jax.numpy as jnp
from jax import lax
from jax.experimental import pallas as pl
from jax.experimental.pallas import tpu as pltpu
from jax.experimental.pallas import tpu_sc as plsc

N = 50000
GRID = 48
C = 128
PG = GRID + 2
SLICE = PG * PG
DN = PG * PG * PG
ODN = GRID * SLICE
MARGIN = 56

NC, NS = 2, 16
NW = NC * NS
NP = 53248
CHUNK = NP // NW
KROWS = CHUNK // 128

_R = (-1, 0, 1)
OFFS = tuple((_R[i] * SLICE + _R[j] * PG + _R[l])
             for i in range(3) for j in range(3) for l in range(3))

_MESH = plsc.VectorSubcoreMesh(core_axis_name="c", subcore_axis_name="s",
                               num_cores=NC, num_subcores=NS)


def _worker_base(chunk):
    wid = lax.axis_index("s") * NC + lax.axis_index("c")
    return wid * chunk


def _compute_idx(cv, idxv, xmul, xoff, cap):
    @pl.loop(0, KROWS)
    def _(j):
        @pl.loop(0, 8)
        def _(l):
            o = j * 128 + l * 16
            x = cv[0, pl.ds(o, 16)]
            y = cv[1, pl.ds(o, 16)]
            z = cv[2, pl.ds(o, 16)]
            idx = (x + xoff) * xmul + (y + 1) * PG + (z + 1)
            idxv[j, pl.ds(l * 16, 16)] = jnp.minimum(idx, cap)


def _scatter_body(coords_hbm, feats_hbm, dense_ref, cv, idxv, fbuf):
    base = _worker_base(CHUNK)
    pltpu.sync_copy(coords_hbm.at[:, pl.ds(base, CHUNK)], cv)
    _compute_idx(cv, idxv, SLICE, 1, DN - 1)

    @pl.loop(0, KROWS)
    def _(j):
        pltpu.sync_copy(feats_hbm.at[pl.ds(base + j * 128, 128)], fbuf)
        pltpu.sync_copy(fbuf, dense_ref.at[idxv.at[j]])


def _gather_body(coords_hbm, od_hbm, out_ref, cv, idxv, gbuf):
    base = _worker_base(CHUNK)
    pltpu.sync_copy(coords_hbm.at[:, pl.ds(base, CHUNK)], cv)
    _compute_idx(cv, idxv, SLICE, 0, ODN - 1)

    @pl.loop(0, KROWS)
    def _(j):
        pltpu.sync_copy(od_hbm.at[idxv.at[j]], gbuf)
        pltpu.sync_copy(gbuf, out_ref.at[pl.ds(base + j * 128, 128)])


_sc_scatter = pl.kernel(
    _scatter_body,
    out_type=(),
    mesh=_MESH,
    scratch_types=[
        pltpu.VMEM((3, CHUNK), jnp.int32),
        pltpu.VMEM((KROWS, 128), jnp.int32),
        pltpu.VMEM((128, C), jnp.float32),
    ],
)

_sc_gather = pl.kernel(
    _gather_body,
    out_type=jax.ShapeDtypeStruct((NP, C), jnp.float32),
    mesh=_MESH,
    scratch_types=[
        pltpu.VMEM((3, CHUNK), jnp.int32),
        pltpu.VMEM((KROWS, 128), jnp.int32),
        pltpu.VMEM((128, C), jnp.float32),
    ],
)


def _conv_body(a_ref, b_ref, c_ref, w_ref, o_ref):
    z = jnp.zeros((MARGIN, C), jnp.float32)
    x = jnp.concatenate([z, a_ref[0], b_ref[0], c_ref[0], z], axis=0)
    acc = None
    for k, d in enumerate(OFFS):
        s = MARGIN + SLICE + d
        t = jnp.dot(x[s:s + SLICE, :], w_ref[k],
                    preferred_element_type=jnp.float32)
        acc = t if acc is None else acc + t
    o_ref[0] = acc


def _conv(dense3, weights):
    blk = pl.BlockSpec((1, SLICE, C), lambda g: (g, 0, 0))
    return pl.pallas_call(
        _conv_body,
        grid=(GRID,),
        in_specs=[
            pl.BlockSpec((1, SLICE, C), lambda g: (g, 0, 0)),
            pl.BlockSpec((1, SLICE, C), lambda g: (g + 1, 0, 0)),
            pl.BlockSpec((1, SLICE, C), lambda g: (g + 2, 0, 0)),
            pl.BlockSpec((27, C, C), lambda g: (0, 0, 0)),
        ],
        out_specs=blk,
        out_shape=jax.ShapeDtypeStruct((GRID, SLICE, C), jnp.float32),
    )(dense3, dense3, dense3, weights)


def kernel(coordinates, features, weights):
    pad = NP - N
    coords_p = jnp.concatenate(
        [coordinates.astype(jnp.int32),
         jnp.full((pad, 3), GRID, jnp.int32)], axis=0)
    coords_t = coords_p.T
    feats_p = jnp.concatenate(
        [features, jnp.zeros((pad, C), features.dtype)], axis=0)

    dense = jax.new_ref(jnp.zeros((DN, C), jnp.float32))
    _sc_scatter(coords_t, feats_p, dense)
    dense3 = dense[...].reshape(PG, SLICE, C)

    out_dense = _conv(dense3, weights)
    od = out_dense.reshape(ODN, C)

    out_p = _sc_gather(coords_t, od)
    return out_p[:N]


# speedup vs baseline: 40.7018x; 1.2754x over previous
import functools

import jax
import jax.numpy as jnp
from jax import lax
from jax.experimental import pallas as pl
from jax.experimental.pallas import tpu as pltpu
from jax.experimental.pallas import tpu_sc as plsc

N = 50000
GRID = 48
C = 128
PG = GRID + 2
SLICE = PG * PG
DN = PG * PG * PG
ODN = GRID * SLICE
MARGIN = 56

NC, NS = 2, 16
NW = NC * NS
NP = 53248
CHUNK = NP // NW
KROWS = CHUNK // 128

_R = (-1, 0, 1)
OFFS = tuple((_R[i] * SLICE + _R[j] * PG + _R[l])
             for i in range(3) for j in range(3) for l in range(3))

_MESH = plsc.VectorSubcoreMesh(core_axis_name="c", subcore_axis_name="s",
                               num_cores=NC, num_subcores=NS)


def _worker_base(chunk):
    wid = lax.axis_index("s") * NC + lax.axis_index("c")
    return wid * chunk


def _compute_idx(cv, idxv, xmul, xoff, cap):
    @pl.loop(0, KROWS)
    def _(j):
        @pl.loop(0, 8)
        def _(l):
            o = j * 128 + l * 16
            x = cv[0, pl.ds(o, 16)]
            y = cv[1, pl.ds(o, 16)]
            z = cv[2, pl.ds(o, 16)]
            idx = (x + xoff) * xmul + (y + 1) * PG + (z + 1)
            idxv[j, pl.ds(l * 16, 16)] = jnp.minimum(idx, cap)


def _scatter_body(coords_hbm, feats_hbm, dense_ref, cv, idxv, fbuf):
    base = _worker_base(CHUNK)
    pltpu.sync_copy(coords_hbm.at[:, pl.ds(base, CHUNK)], cv)
    _compute_idx(cv, idxv, SLICE, 1, DN - 1)

    @pl.loop(0, KROWS)
    def _(j):
        pltpu.sync_copy(feats_hbm.at[pl.ds(base + j * 128, 128)], fbuf)
        pltpu.sync_copy(fbuf, dense_ref.at[idxv.at[j]])


def _gather_body(coords_hbm, od_hbm, out_ref, cv, idxv, gbuf):
    base = _worker_base(CHUNK)
    pltpu.sync_copy(coords_hbm.at[:, pl.ds(base, CHUNK)], cv)
    _compute_idx(cv, idxv, SLICE, 0, ODN - 1)

    @pl.loop(0, KROWS)
    def _(j):
        pltpu.sync_copy(od_hbm.at[idxv.at[j]], gbuf)
        pltpu.sync_copy(gbuf, out_ref.at[pl.ds(base + j * 128, 128)])


_sc_scatter = pl.kernel(
    _scatter_body,
    out_type=(),
    mesh=_MESH,
    scratch_types=[
        pltpu.VMEM((3, CHUNK), jnp.int32),
        pltpu.VMEM((KROWS, 128), jnp.int32),
        pltpu.VMEM((128, C), jnp.float32),
    ],
)

_sc_gather = pl.kernel(
    _gather_body,
    out_type=jax.ShapeDtypeStruct((NP, C), jnp.float32),
    mesh=_MESH,
    scratch_types=[
        pltpu.VMEM((3, CHUNK), jnp.int32),
        pltpu.VMEM((KROWS, 128), jnp.int32),
        pltpu.VMEM((128, C), jnp.float32),
    ],
)


OFFS28 = OFFS + (0,)


def _conv_body(a_ref, b_ref, c_ref, w_ref, o_ref):
    z = jnp.zeros((MARGIN, C), jnp.bfloat16)
    x = jnp.concatenate(
        [z, a_ref[0].astype(jnp.bfloat16), b_ref[0].astype(jnp.bfloat16),
         c_ref[0].astype(jnp.bfloat16), z], axis=0)
    acc = None
    for p in range(14):
        d1, d2 = OFFS28[2 * p], OFFS28[2 * p + 1]
        s1 = MARGIN + SLICE + d1
        s2 = MARGIN + SLICE + d2
        xp = jnp.concatenate(
            [x[s1:s1 + SLICE, :], x[s2:s2 + SLICE, :]], axis=1)
        t = jnp.dot(xp, w_ref[p], preferred_element_type=jnp.float32)
        acc = t if acc is None else acc + t
    o_ref[0] = acc


def _conv(dense3, weights):
    wb = jnp.concatenate(
        [weights, jnp.zeros((1, C, C), weights.dtype)], axis=0)
    wpair = wb.reshape(14, 2 * C, C)
    return pl.pallas_call(
        _conv_body,
        grid=(GRID,),
        in_specs=[
            pl.BlockSpec((1, SLICE, C), lambda g: (g, 0, 0)),
            pl.BlockSpec((1, SLICE, C), lambda g: (g + 1, 0, 0)),
            pl.BlockSpec((1, SLICE, C), lambda g: (g + 2, 0, 0)),
            pl.BlockSpec((14, 2 * C, C), lambda g: (0, 0, 0)),
        ],
        out_specs=pl.BlockSpec((1, SLICE, C), lambda g: (g, 0, 0)),
        out_shape=jax.ShapeDtypeStruct((GRID, SLICE, C), jnp.float32),
        compiler_params=pltpu.CompilerParams(
            dimension_semantics=("parallel",)),
    )(dense3, dense3, dense3, wpair)


def kernel(coordinates, features, weights):
    pad = NP - N
    coords_p = jnp.concatenate(
        [coordinates.astype(jnp.int32),
         jnp.full((pad, 3), GRID, jnp.int32)], axis=0)
    coords_t = coords_p.T
    feats_p = jnp.concatenate(
        [features, jnp.zeros((pad, C), features.dtype)], axis=0)

    dense = jax.new_ref(jnp.zeros((DN, C), jnp.float32))
    _sc_scatter(coords_t, feats_p, dense)
    dense3 = dense[...].reshape(PG, SLICE, C)

    out_dense = _conv(dense3, weights.astype(jnp.bfloat16))
    od = out_dense.reshape(ODN, C)

    out_p = _sc_gather(coords_t, od)
    return out_p[:N]


# speedup vs baseline: 46.4580x; 1.1414x over previous
import functools

import jax
import jax.numpy as jnp
from jax import lax
from jax.experimental import pallas as pl
from jax.experimental.pallas import tpu as pltpu
from jax.experimental.pallas import tpu_sc as plsc

N = 50000
GRID = 48
C = 128
PG = GRID + 2
SLICE = 2504
DN = PG * SLICE
ODN = GRID * SLICE
MARGIN = 56

NC, NS = 2, 16
NW = NC * NS
NP = 53248
CHUNK = NP // NW
KROWS = CHUNK // 128
NBUF = 3

_R = (-1, 0, 1)
OFFS = tuple((_R[i] * SLICE + _R[j] * PG + _R[l])
             for i in range(3) for j in range(3) for l in range(3))
OFFS28 = OFFS + (0,)

_MESH = plsc.VectorSubcoreMesh(core_axis_name="c", subcore_axis_name="s",
                               num_cores=NC, num_subcores=NS)

_SC_SCRATCH = [
    pltpu.VMEM((3, CHUNK), jnp.int32),
    pltpu.VMEM((KROWS, 128), jnp.int32),
    pltpu.VMEM((NBUF, 128, C), jnp.float32),
] + [pltpu.SemaphoreType.DMA] * (2 * NBUF)


def _worker_base():
    wid = lax.axis_index("s") * NC + lax.axis_index("c")
    return wid * CHUNK


def _compute_idx(cv, idxv, xmul, xoff, cap):
    @pl.loop(0, KROWS)
    def _(j):
        @pl.loop(0, 8)
        def _(l):
            o = j * 128 + l * 16
            x = cv[0, pl.ds(o, 16)]
            y = cv[1, pl.ds(o, 16)]
            z = cv[2, pl.ds(o, 16)]
            idx = (x + xoff) * SLICE + (y + 1) * PG + (z + 1)
            idxv[j, pl.ds(l * 16, 16)] = jnp.minimum(idx, cap)


def _pipeline(load, store, sems):
    sl, ss = sems[:NBUF], sems[NBUF:]
    dl = [None] * KROWS
    ds = [None] * KROWS
    dl[0] = load(0, sl[0])
    for j in range(KROWS):
        dl[j].wait()
        ds[j] = store(j, ss[j % NBUF])
        if j + 1 < KROWS:
            if j >= NBUF - 1:
                ds[j - NBUF + 1].wait()
            dl[j + 1] = load(j + 1, sl[(j + 1) % NBUF])
    for j in range(KROWS - NBUF, KROWS):
        ds[j].wait()


def _scatter_body(coords_hbm, feats_hbm, dense_ref, cv, idxv, fb, *sems):
    base = _worker_base()
    pltpu.sync_copy(coords_hbm.at[:, pl.ds(base, CHUNK)], cv)
    _compute_idx(cv, idxv, SLICE, 1, DN - 1)

    def load(j, sem):
        return pltpu.async_copy(
            feats_hbm.at[pl.ds(base + j * 128, 128)], fb.at[j % NBUF], sem)

    def store(j, sem):
        return pltpu.async_copy(
            fb.at[j % NBUF], dense_ref.at[idxv.at[j]], sem)

    _pipeline(load, store, sems)


def _gather_body(coords_hbm, od_hbm, out_ref, cv, idxv, gb, *sems):
    base = _worker_base()
    pltpu.sync_copy(coords_hbm.at[:, pl.ds(base, CHUNK)], cv)
    _compute_idx(cv, idxv, SLICE, 0, ODN - 1)

    def load(j, sem):
        return pltpu.async_copy(
            od_hbm.at[idxv.at[j]], gb.at[j % NBUF], sem)

    def store(j, sem):
        return pltpu.async_copy(
            gb.at[j % NBUF], out_ref.at[pl.ds(base + j * 128, 128)], sem)

    _pipeline(load, store, sems)


_sc_scatter = pl.kernel(
    _scatter_body, out_type=(), mesh=_MESH, scratch_types=_SC_SCRATCH)

_sc_gather = pl.kernel(
    _gather_body,
    out_type=jax.ShapeDtypeStruct((NP, C), jnp.float32),
    mesh=_MESH, scratch_types=_SC_SCRATCH)


def _conv_body(a_ref, b_ref, c_ref, w_ref, o_ref):
    z = jnp.zeros((MARGIN, C), jnp.bfloat16)
    x = jnp.concatenate(
        [z, a_ref[...].astype(jnp.bfloat16), b_ref[...].astype(jnp.bfloat16),
         c_ref[...].astype(jnp.bfloat16), z], axis=0)
    acc = None
    for p in range(14):
        d1, d2 = OFFS28[2 * p], OFFS28[2 * p + 1]
        s1 = MARGIN + SLICE + d1
        s2 = MARGIN + SLICE + d2
        xp = jnp.concatenate(
            [x[s1:s1 + SLICE, :], x[s2:s2 + SLICE, :]], axis=1)
        t = jnp.dot(xp, w_ref[p], preferred_element_type=jnp.float32)
        acc = t if acc is None else acc + t
    o_ref[...] = acc


def _conv(dense, weights):
    wb = jnp.concatenate(
        [weights, jnp.zeros((1, C, C), weights.dtype)], axis=0)
    wpair = wb.reshape(14, 2 * C, C)
    return pl.pallas_call(
        _conv_body,
        grid=(GRID,),
        in_specs=[
            pl.BlockSpec((SLICE, C), lambda g: (g, 0)),
            pl.BlockSpec((SLICE, C), lambda g: (g + 1, 0)),
            pl.BlockSpec((SLICE, C), lambda g: (g + 2, 0)),
            pl.BlockSpec((14, 2 * C, C), lambda g: (0, 0, 0)),
        ],
        out_specs=pl.BlockSpec((SLICE, C), lambda g: (g, 0)),
        out_shape=jax.ShapeDtypeStruct((ODN, C), jnp.float32),
        compiler_params=pltpu.CompilerParams(
            dimension_semantics=("parallel",)),
    )(dense, dense, dense, wpair)


def kernel(coordinates, features, weights):
    pad = NP - N
    coords_p = jnp.concatenate(
        [coordinates.astype(jnp.int32),
         jnp.full((pad, 3), GRID, jnp.int32)], axis=0)
    coords_t = coords_p.T
    feats_p = jnp.concatenate(
        [features, jnp.zeros((pad, C), features.dtype)], axis=0)

    dense = jax.new_ref(jnp.zeros((DN, C), jnp.float32))
    _sc_scatter(coords_t, feats_p, dense)

    od = _conv(dense[...], weights.astype(jnp.bfloat16))

    out_p = _sc_gather(coords_t, od)
    return out_p[:N]


# speedup vs baseline: 46.5335x; 1.0016x over previous
import functools

import jax
import jax.numpy as jnp
from jax import lax
from jax.experimental import pallas as pl
from jax.experimental.pallas import tpu as pltpu
from jax.experimental.pallas import tpu_sc as plsc

N = 50000
GRID = 48
C = 128
PG = GRID + 2
SLICE = 2504
DN = PG * SLICE
ODN = GRID * SLICE
MARGIN = 56

NC, NS = 2, 16
NW = NC * NS
NP = 53248
CHUNK = NP // NW
KROWS = CHUNK // 128
NBUF = 6

_R = (-1, 0, 1)
OFFS = tuple((_R[i] * SLICE + _R[j] * PG + _R[l])
             for i in range(3) for j in range(3) for l in range(3))
OFFS28 = OFFS + (0,)

_MESH = plsc.VectorSubcoreMesh(core_axis_name="c", subcore_axis_name="s",
                               num_cores=NC, num_subcores=NS)

_SC_SCRATCH = [
    pltpu.VMEM((3, CHUNK), jnp.int32),
    pltpu.VMEM((KROWS, 128), jnp.int32),
    pltpu.VMEM((NBUF, 128, C), jnp.float32),
] + [pltpu.SemaphoreType.DMA] * (2 * NBUF)


def _worker_base():
    wid = lax.axis_index("s") * NC + lax.axis_index("c")
    return wid * CHUNK


def _compute_idx(cv, idxv, xmul, xoff, cap):
    @pl.loop(0, KROWS)
    def _(j):
        @pl.loop(0, 8)
        def _(l):
            o = j * 128 + l * 16
            x = cv[0, pl.ds(o, 16)]
            y = cv[1, pl.ds(o, 16)]
            z = cv[2, pl.ds(o, 16)]
            idx = (x + xoff) * SLICE + (y + 1) * PG + (z + 1)
            idxv[j, pl.ds(l * 16, 16)] = jnp.minimum(idx, cap)


def _pipeline(load, store, sems):
    sl, ss = sems[:NBUF], sems[NBUF:]
    dl = [None] * KROWS
    ds = [None] * KROWS
    dl[0] = load(0, sl[0])
    for j in range(KROWS):
        dl[j].wait()
        ds[j] = store(j, ss[j % NBUF])
        if j + 1 < KROWS:
            if j >= NBUF - 1:
                ds[j - NBUF + 1].wait()
            dl[j + 1] = load(j + 1, sl[(j + 1) % NBUF])
    for j in range(KROWS - NBUF, KROWS):
        ds[j].wait()


def _scatter_body(coords_hbm, feats_hbm, dense_ref, cv, idxv, fb, *sems):
    base = _worker_base()
    pltpu.sync_copy(coords_hbm.at[:, pl.ds(base, CHUNK)], cv)
    _compute_idx(cv, idxv, SLICE, 1, DN - 1)

    def load(j, sem):
        return pltpu.async_copy(
            feats_hbm.at[pl.ds(base + j * 128, 128)], fb.at[j % NBUF], sem)

    def store(j, sem):
        return pltpu.async_copy(
            fb.at[j % NBUF], dense_ref.at[idxv.at[j]], sem)

    _pipeline(load, store, sems)


def _gather_body(coords_hbm, od_hbm, out_ref, cv, idxv, gb, *sems):
    base = _worker_base()
    pltpu.sync_copy(coords_hbm.at[:, pl.ds(base, CHUNK)], cv)
    _compute_idx(cv, idxv, SLICE, 0, ODN - 1)

    def load(j, sem):
        return pltpu.async_copy(
            od_hbm.at[idxv.at[j]], gb.at[j % NBUF], sem)

    def store(j, sem):
        return pltpu.async_copy(
            gb.at[j % NBUF], out_ref.at[pl.ds(base + j * 128, 128)], sem)

    _pipeline(load, store, sems)


_sc_scatter = pl.kernel(
    _scatter_body, out_type=(), mesh=_MESH, scratch_types=_SC_SCRATCH)

_sc_gather = pl.kernel(
    _gather_body,
    out_type=jax.ShapeDtypeStruct((NP, C), jnp.float32),
    mesh=_MESH, scratch_types=_SC_SCRATCH)


def _conv_body(a_ref, b_ref, c_ref, w_ref, o_ref):
    z = jnp.zeros((MARGIN, C), jnp.bfloat16)
    x = jnp.concatenate(
        [z, a_ref[...].astype(jnp.bfloat16), b_ref[...].astype(jnp.bfloat16),
         c_ref[...].astype(jnp.bfloat16), z], axis=0)
    acc = None
    for p in range(14):
        d1, d2 = OFFS28[2 * p], OFFS28[2 * p + 1]
        s1 = MARGIN + SLICE + d1
        s2 = MARGIN + SLICE + d2
        xp = jnp.concatenate(
            [x[s1:s1 + SLICE, :], x[s2:s2 + SLICE, :]], axis=1)
        t = jnp.dot(xp, w_ref[p], preferred_element_type=jnp.float32)
        acc = t if acc is None else acc + t
    o_ref[...] = acc


def _conv(dense, weights):
    wb = jnp.concatenate(
        [weights, jnp.zeros((1, C, C), weights.dtype)], axis=0)
    wpair = wb.reshape(14, 2 * C, C)
    return pl.pallas_call(
        _conv_body,
        grid=(GRID,),
        in_specs=[
            pl.BlockSpec((SLICE, C), lambda g: (g, 0)),
            pl.BlockSpec((SLICE, C), lambda g: (g + 1, 0)),
            pl.BlockSpec((SLICE, C), lambda g: (g + 2, 0)),
            pl.BlockSpec((14, 2 * C, C), lambda g: (0, 0, 0)),
        ],
        out_specs=pl.BlockSpec((SLICE, C), lambda g: (g, 0)),
        out_shape=jax.ShapeDtypeStruct((ODN, C), jnp.float32),
        compiler_params=pltpu.CompilerParams(
            dimension_semantics=("parallel",)),
    )(dense, dense, dense, wpair)


def kernel(coordinates, features, weights):
    pad = NP - N
    coords_p = jnp.concatenate(
        [coordinates.astype(jnp.int32),
         jnp.full((pad, 3), GRID, jnp.int32)], axis=0)
    coords_t = coords_p.T
    feats_p = jnp.concatenate(
        [features, jnp.zeros((pad, C), features.dtype)], axis=0)

    dense = jax.new_ref(jnp.zeros((DN, C), jnp.float32))
    _sc_scatter(coords_t, feats_p, dense)

    od = _conv(dense[...], weights.astype(jnp.bfloat16))

    out_p = _sc_gather(coords_t, od)
    return out_p[:N]


# speedup vs baseline: 47.0265x; 1.0106x over previous
import functools

import jax
import jax.numpy as jnp
from jax import lax
from jax.experimental import pallas as pl
from jax.experimental.pallas import tpu as pltpu
from jax.experimental.pallas import tpu_sc as plsc

N = 50000
GRID = 48
C = 128
PG = GRID + 2
YS = 50
SLICE = 2504
DN = PG * SLICE
ODN = GRID * SLICE
MARGIN = 56

NC, NS = 2, 16
NW = NC * NS
NP = 53248
CHUNK = NP // NW
KROWS = CHUNK // 128
NBUF = 6

_R = (-1, 0, 1)
OFFS = tuple((_R[i] * SLICE + _R[j] * YS + _R[l])
             for i in range(3) for j in range(3) for l in range(3))
OFFS28 = OFFS + (0,)

_MESH = plsc.VectorSubcoreMesh(core_axis_name="c", subcore_axis_name="s",
                               num_cores=NC, num_subcores=NS)

_SC_SCRATCH = [
    pltpu.VMEM((3, CHUNK), jnp.int32),
    pltpu.VMEM((KROWS, 128), jnp.int32),
    pltpu.VMEM((NBUF, 128, C), jnp.float32),
] + [pltpu.SemaphoreType.DMA] * (2 * NBUF)


def _worker_base():
    wid = lax.axis_index("s") * NC + lax.axis_index("c")
    return wid * CHUNK


def _compute_idx(cv, idxv, xmul, xoff, cap):
    @pl.loop(0, KROWS)
    def _(j):
        @pl.loop(0, 8)
        def _(l):
            o = j * 128 + l * 16
            x = cv[0, pl.ds(o, 16)]
            y = cv[1, pl.ds(o, 16)]
            z = cv[2, pl.ds(o, 16)]
            idx = (x + xoff) * SLICE + (y + 1) * YS + (z + 1)
            idxv[j, pl.ds(l * 16, 16)] = jnp.minimum(idx, cap)


def _pipeline(load, store, sems):
    sl, ss = sems[:NBUF], sems[NBUF:]
    dl = [None] * KROWS
    ds = [None] * KROWS
    dl[0] = load(0, sl[0])
    for j in range(KROWS):
        dl[j].wait()
        ds[j] = store(j, ss[j % NBUF])
        if j + 1 < KROWS:
            if j >= NBUF - 1:
                ds[j - NBUF + 1].wait()
            dl[j + 1] = load(j + 1, sl[(j + 1) % NBUF])
    for j in range(KROWS - NBUF, KROWS):
        ds[j].wait()


def _scatter_body(coords_hbm, feats_hbm, dense_ref, cv, idxv, fb, *sems):
    base = _worker_base()
    pltpu.sync_copy(coords_hbm.at[:, pl.ds(base, CHUNK)], cv)
    _compute_idx(cv, idxv, SLICE, 1, DN - 1)

    def load(j, sem):
        return pltpu.async_copy(
            feats_hbm.at[pl.ds(base + j * 128, 128)], fb.at[j % NBUF], sem)

    def store(j, sem):
        return pltpu.async_copy(
            fb.at[j % NBUF], dense_ref.at[idxv.at[j]], sem)

    _pipeline(load, store, sems)


def _gather_body(coords_hbm, od_hbm, out_ref, cv, idxv, gb, *sems):
    base = _worker_base()
    pltpu.sync_copy(coords_hbm.at[:, pl.ds(base, CHUNK)], cv)
    _compute_idx(cv, idxv, SLICE, 0, ODN - 1)

    def load(j, sem):
        return pltpu.async_copy(
            od_hbm.at[idxv.at[j]], gb.at[j % NBUF], sem)

    def store(j, sem):
        return pltpu.async_copy(
            gb.at[j % NBUF], out_ref.at[pl.ds(base + j * 128, 128)], sem)

    _pipeline(load, store, sems)


_sc_scatter = pl.kernel(
    _scatter_body, out_type=(), mesh=_MESH, scratch_types=_SC_SCRATCH)

_sc_gather = pl.kernel(
    _gather_body,
    out_type=jax.ShapeDtypeStruct((NP, C), jnp.float32),
    mesh=_MESH, scratch_types=_SC_SCRATCH)


CBS = 2
CROWS = CBS * SLICE


def _conv_body(*refs):
    in_refs, w_ref, o_ref = refs[:CBS + 2], refs[CBS + 2], refs[CBS + 3]
    z = jnp.zeros((MARGIN, C), jnp.bfloat16)
    x = jnp.concatenate(
        [z] + [r[...].astype(jnp.bfloat16) for r in in_refs] + [z], axis=0)
    acc = None
    for p in range(14):
        d1, d2 = OFFS28[2 * p], OFFS28[2 * p + 1]
        s1 = MARGIN + SLICE + d1
        s2 = MARGIN + SLICE + d2
        xp = jnp.concatenate(
            [x[s1:s1 + CROWS, :], x[s2:s2 + CROWS, :]], axis=1)
        t = jnp.dot(xp, w_ref[p], preferred_element_type=jnp.float32)
        acc = t if acc is None else acc + t
    o_ref[...] = acc


def _conv(dense, weights):
    wb = jnp.concatenate(
        [weights, jnp.zeros((1, C, C), weights.dtype)], axis=0)
    wpair = wb.reshape(14, 2 * C, C)
    in_specs = [
        pl.BlockSpec((SLICE, C),
                     functools.partial(lambda i, g: (CBS * g + i, 0), i))
        for i in range(CBS + 2)
    ]
    in_specs.append(pl.BlockSpec((14, 2 * C, C), lambda g: (0, 0, 0)))
    return pl.pallas_call(
        _conv_body,
        grid=(GRID // CBS,),
        in_specs=in_specs,
        out_specs=pl.BlockSpec((CROWS, C), lambda g: (g, 0)),
        out_shape=jax.ShapeDtypeStruct((ODN, C), jnp.float32),
        compiler_params=pltpu.CompilerParams(
            dimension_semantics=("parallel",)),
    )(*([dense] * (CBS + 2) + [wpair]))


def kernel(coordinates, features, weights):
    pad = NP - N
    coords_p = jnp.concatenate(
        [coordinates.astype(jnp.int32),
         jnp.full((pad, 3), GRID, jnp.int32)], axis=0)
    coords_t = coords_p.T
    feats_p = jnp.concatenate(
        [features, jnp.zeros((pad, C), features.dtype)], axis=0)

    dense = jax.new_ref(jnp.zeros((DN, C), jnp.float32))
    _sc_scatter(coords_t, feats_p, dense)

    od = _conv(dense[...], weights.astype(jnp.bfloat16))

    out_p = _sc_gather(coords_t, od)
    return out_p[:N]


# speedup vs baseline: 87.4578x; 1.8598x over previous
import functools

import jax
import jax.numpy as jnp
from jax import lax
from jax.experimental import pallas as pl
from jax.experimental.pallas import tpu as pltpu
from jax.experimental.pallas import tpu_sc as plsc

N = 50000
GRID = 48
C = 128
PG = GRID + 2
YS = 50
SLICE = 2504
DN = PG * SLICE
ODN = GRID * SLICE
MARGIN = 56

NC, NS = 2, 16
NW = NC * NS
NP = 53248
CHUNK = NP // NW
KROWS = CHUNK // 128
NBUF = 6
NFULL = N // CHUNK
TBASE = NFULL * CHUNK
TREM = N - TBASE

_R = (-1, 0, 1)
OFFS = tuple((_R[i] * SLICE + _R[j] * YS + _R[l])
             for i in range(3) for j in range(3) for l in range(3))
OFFS28 = OFFS + (0,)

_MESH = plsc.VectorSubcoreMesh(core_axis_name="c", subcore_axis_name="s",
                               num_cores=NC, num_subcores=NS)

_SC_SCRATCH = [
    pltpu.VMEM((3, CHUNK), jnp.int32),
    pltpu.VMEM((KROWS, 128), jnp.int32),
    pltpu.VMEM((NBUF, 128, C), jnp.float32),
    pltpu.VMEM((TREM,), jnp.int32),
    pltpu.VMEM((TREM, C), jnp.float32),
] + [pltpu.SemaphoreType.DMA] * (2 * NBUF)


def _worker_id():
    return lax.axis_index("s") * NC + lax.axis_index("c")


def _tail_idx(cv, idxp, xoff, cap):
    @pl.loop(0, TREM // 16)
    def _(l):
        o = l * 16
        x = cv[0, pl.ds(o, 16)]
        y = cv[1, pl.ds(o, 16)]
        z = cv[2, pl.ds(o, 16)]
        idx = (x + xoff) * SLICE + (y + 1) * YS + (z + 1)
        idxp[pl.ds(o, 16)] = jnp.minimum(idx, cap)


def _compute_idx(cv, idxv, xmul, xoff, cap):
    @pl.loop(0, KROWS)
    def _(j):
        @pl.loop(0, 8)
        def _(l):
            o = j * 128 + l * 16
            x = cv[0, pl.ds(o, 16)]
            y = cv[1, pl.ds(o, 16)]
            z = cv[2, pl.ds(o, 16)]
            idx = (x + xoff) * SLICE + (y + 1) * YS + (z + 1)
            idxv[j, pl.ds(l * 16, 16)] = jnp.minimum(idx, cap)


def _pipeline(load, store, sems):
    sl, ss = sems[:NBUF], sems[NBUF:]
    dl = [None] * KROWS
    ds = [None] * KROWS
    dl[0] = load(0, sl[0])
    for j in range(KROWS):
        dl[j].wait()
        ds[j] = store(j, ss[j % NBUF])
        if j + 1 < KROWS:
            if j >= NBUF - 1:
                ds[j - NBUF + 1].wait()
            dl[j + 1] = load(j + 1, sl[(j + 1) % NBUF])
    for j in range(KROWS - NBUF, KROWS):
        ds[j].wait()


def _scatter_body(coords_hbm, feats_hbm, dense_ref, cv, idxv, fb, idxp, fbp,
                  *sems):
    wid = _worker_id()
    base = wid * CHUNK
    pltpu.sync_copy(coords_hbm.at[:, pl.ds(base, CHUNK)], cv)

    @pl.when(wid < NFULL)
    def _():
        _compute_idx(cv, idxv, SLICE, 1, DN - 1)

        def load(j, sem):
            return pltpu.async_copy(
                feats_hbm.at[pl.ds(base + j * 128, 128)], fb.at[j % NBUF],
                sem)

        def store(j, sem):
            return pltpu.async_copy(
                fb.at[j % NBUF], dense_ref.at[idxv.at[j]], sem)

        _pipeline(load, store, sems)

    @pl.when(wid == NFULL)
    def _():
        _tail_idx(cv, idxp, 1, DN - 1)
        pltpu.sync_copy(feats_hbm.at[pl.ds(TBASE, TREM)], fbp)
        pltpu.sync_copy(fbp, dense_ref.at[idxp])


def _gather_body(coords_hbm, od_hbm, out_ref, cv, idxv, gb, idxp, gbp,
                 *sems):
    wid = _worker_id()
    base = wid * CHUNK
    pltpu.sync_copy(coords_hbm.at[:, pl.ds(base, CHUNK)], cv)

    @pl.when(wid < NFULL)
    def _():
        _compute_idx(cv, idxv, SLICE, 0, ODN - 1)

        def load(j, sem):
            return pltpu.async_copy(
                od_hbm.at[idxv.at[j]], gb.at[j % NBUF], sem)

        def store(j, sem):
            return pltpu.async_copy(
                gb.at[j % NBUF], out_ref.at[pl.ds(base + j * 128, 128)], sem)

        _pipeline(load, store, sems)

    @pl.when(wid == NFULL)
    def _():
        _tail_idx(cv, idxp, 0, ODN - 1)
        pltpu.sync_copy(od_hbm.at[idxp], gbp)
        pltpu.sync_copy(gbp, out_ref.at[pl.ds(TBASE, TREM)])


_sc_scatter = pl.kernel(
    _scatter_body, out_type=(), mesh=_MESH, scratch_types=_SC_SCRATCH)

_sc_gather = pl.kernel(
    _gather_body,
    out_type=jax.ShapeDtypeStruct((N, C), jnp.float32),
    mesh=_MESH, scratch_types=_SC_SCRATCH)


CBS = 2
CROWS = CBS * SLICE


def _conv_body(*refs):
    in_refs, w_ref, o_ref = refs[:CBS + 2], refs[CBS + 2], refs[CBS + 3]
    z = jnp.zeros((MARGIN, C), jnp.bfloat16)
    x = jnp.concatenate(
        [z] + [r[...].astype(jnp.bfloat16) for r in in_refs] + [z], axis=0)
    acc = None
    for p in range(14):
        d1, d2 = OFFS28[2 * p], OFFS28[2 * p + 1]
        s1 = MARGIN + SLICE + d1
        s2 = MARGIN + SLICE + d2
        xp = jnp.concatenate(
            [x[s1:s1 + CROWS, :], x[s2:s2 + CROWS, :]], axis=1)
        t = jnp.dot(xp, w_ref[p], preferred_element_type=jnp.float32)
        acc = t if acc is None else acc + t
    o_ref[...] = acc


def _conv(dense, weights):
    wb = jnp.concatenate(
        [weights, jnp.zeros((1, C, C), weights.dtype)], axis=0)
    wpair = wb.reshape(14, 2 * C, C)
    in_specs = [
        pl.BlockSpec((SLICE, C),
                     functools.partial(lambda i, g: (CBS * g + i, 0), i))
        for i in range(CBS + 2)
    ]
    in_specs.append(pl.BlockSpec((14, 2 * C, C), lambda g: (0, 0, 0)))
    return pl.pallas_call(
        _conv_body,
        grid=(GRID // CBS,),
        in_specs=in_specs,
        out_specs=pl.BlockSpec((CROWS, C), lambda g: (g, 0)),
        out_shape=jax.ShapeDtypeStruct((ODN, C), jnp.float32),
        compiler_params=pltpu.CompilerParams(
            dimension_semantics=("parallel",)),
    )(*([dense] * (CBS + 2) + [wpair]))


def kernel(coordinates, features, weights):
    pad = NP - N
    coords_p = jnp.concatenate(
        [coordinates.astype(jnp.int32),
         jnp.full((pad, 3), GRID, jnp.int32)], axis=0)
    coords_t = coords_p.T

    dense = jax.new_ref(jnp.zeros((DN, C), jnp.float32))
    _sc_scatter(coords_t, features, dense)

    od = _conv(dense[...], weights.astype(jnp.bfloat16))

    return _sc_gather(coords_t, od)


# speedup vs baseline: 87.6264x; 1.0019x over previous
import functools

import jax
import jax.numpy as jnp
from jax import lax
from jax.experimental import pallas as pl
from jax.experimental.pallas import tpu as pltpu
from jax.experimental.pallas import tpu_sc as plsc

N = 50000
GRID = 48
C = 128
PG = GRID + 2
YS = 50
SLICE = 2504
DN = PG * SLICE
ODN = GRID * SLICE
MARGIN = 56

NC, NS = 2, 16
NW = NC * NS
NP = 53248
CHUNK = NP // NW
KROWS = CHUNK // 128
NBUF = 6
NFULL = N // CHUNK
TBASE = NFULL * CHUNK
TREM = N - TBASE

_R = (-1, 0, 1)
OFFS = tuple((_R[i] * SLICE + _R[j] * YS + _R[l])
             for i in range(3) for j in range(3) for l in range(3))
OFFS28 = OFFS + (0,)

_MESH = plsc.VectorSubcoreMesh(core_axis_name="c", subcore_axis_name="s",
                               num_cores=NC, num_subcores=NS)

_SC_SCRATCH = [
    pltpu.VMEM((3, CHUNK), jnp.int32),
    pltpu.VMEM((KROWS, 128), jnp.int32),
    pltpu.VMEM((NBUF, 128, C), jnp.float32),
    pltpu.VMEM((TREM,), jnp.int32),
    pltpu.VMEM((TREM, C), jnp.float32),
] + [pltpu.SemaphoreType.DMA] * (2 * NBUF)


def _worker_id():
    return lax.axis_index("s") * NC + lax.axis_index("c")


def _tail_idx(cv, idxp, xoff, cap):
    @pl.loop(0, TREM // 16)
    def _(l):
        o = l * 16
        x = cv[0, pl.ds(o, 16)]
        y = cv[1, pl.ds(o, 16)]
        z = cv[2, pl.ds(o, 16)]
        idx = (x + xoff) * SLICE + (y + 1) * YS + (z + 1)
        idxp[pl.ds(o, 16)] = jnp.minimum(idx, cap)


def _compute_idx(cv, idxv, xoff, cap):
    @pl.loop(0, KROWS)
    def _(j):
        @pl.loop(0, 8)
        def _(l):
            o = j * 128 + l * 16
            x = cv[0, pl.ds(o, 16)]
            y = cv[1, pl.ds(o, 16)]
            z = cv[2, pl.ds(o, 16)]
            idx = (x + xoff) * SLICE + (y + 1) * YS + (z + 1)
            idxv[j, pl.ds(l * 16, 16)] = jnp.minimum(idx, cap)


def _pipeline(load, store, sems):
    sl, ss = sems[:NBUF], sems[NBUF:]
    dl = [None] * KROWS
    ds = [None] * KROWS
    dl[0] = load(0, sl[0])
    for j in range(KROWS):
        dl[j].wait()
        ds[j] = store(j, ss[j % NBUF])
        if j + 1 < KROWS:
            if j >= NBUF - 1:
                ds[j - NBUF + 1].wait()
            dl[j + 1] = load(j + 1, sl[(j + 1) % NBUF])
    for j in range(KROWS - NBUF, KROWS):
        ds[j].wait()


def _scatter_body(coords_hbm, feats_hbm, dense_ref, cv, idxv, fb, idxp, fbp,
                  *sems):
    wid = _worker_id()
    base = wid * CHUNK
    pltpu.sync_copy(coords_hbm.at[:, pl.ds(base, CHUNK)], cv)

    @pl.when(wid < NFULL)
    def _():
        _compute_idx(cv, idxv, 1, DN - 1)

        def load(j, sem):
            return pltpu.async_copy(
                feats_hbm.at[pl.ds(base + j * 128, 128)], fb.at[j % NBUF],
                sem)

        def store(j, sem):
            return pltpu.async_copy(
                fb.at[j % NBUF], dense_ref.at[idxv.at[j]], sem)

        _pipeline(load, store, sems)

    @pl.when(wid == NFULL)
    def _():
        _tail_idx(cv, idxp, 1, DN - 1)
        pltpu.sync_copy(feats_hbm.at[pl.ds(TBASE, TREM)], fbp)
        pltpu.sync_copy(fbp, dense_ref.at[idxp])


def _gather_body(coords_hbm, od_hbm, out_ref, cv, idxv, gb, idxp, gbp,
                 *sems):
    wid = _worker_id()
    base = wid * CHUNK
    pltpu.sync_copy(coords_hbm.at[:, pl.ds(base, CHUNK)], cv)

    @pl.when(wid < NFULL)
    def _():
        _compute_idx(cv, idxv, 0, ODN - 1)

        def load(j, sem):
            return pltpu.async_copy(
                od_hbm.at[idxv.at[j]], gb.at[j % NBUF], sem)

        def store(j, sem):
            return pltpu.async_copy(
                gb.at[j % NBUF], out_ref.at[pl.ds(base + j * 128, 128)], sem)

        _pipeline(load, store, sems)

    @pl.when(wid == NFULL)
    def _():
        _tail_idx(cv, idxp, 0, ODN - 1)
        pltpu.sync_copy(od_hbm.at[idxp], gbp)
        pltpu.sync_copy(gbp, out_ref.at[pl.ds(TBASE, TREM)])


_sc_scatter = pl.kernel(
    _scatter_body, out_type=(), mesh=_MESH, scratch_types=_SC_SCRATCH)

_sc_gather = pl.kernel(
    _gather_body,
    out_type=jax.ShapeDtypeStruct((N, C), jnp.float32),
    mesh=_MESH, scratch_types=_SC_SCRATCH)


CBS = 2
CROWS = CBS * SLICE


def _conv_body(*refs):
    in_refs, w_ref, o_ref = refs[:CBS + 2], refs[CBS + 2], refs[CBS + 3]
    z = jnp.zeros((MARGIN, C), jnp.bfloat16)
    x = jnp.concatenate(
        [z] + [r[...].astype(jnp.bfloat16) for r in in_refs] + [z], axis=0)
    acc = None
    for p in range(14):
        d1, d2 = OFFS28[2 * p], OFFS28[2 * p + 1]
        s1 = MARGIN + SLICE + d1
        s2 = MARGIN + SLICE + d2
        xp = jnp.concatenate(
            [x[s1:s1 + CROWS, :], x[s2:s2 + CROWS, :]], axis=1)
        t = jnp.dot(xp, w_ref[p], preferred_element_type=jnp.float32)
        acc = t if acc is None else acc + t
    o_ref[...] = acc


def _conv(dense, weights):
    wb = jnp.concatenate(
        [weights, jnp.zeros((1, C, C), weights.dtype)], axis=0)
    wpair = wb.reshape(14, 2 * C, C)
    in_specs = [
        pl.BlockSpec((SLICE, C),
                     functools.partial(lambda i, g: (CBS * g + i, 0), i))
        for i in range(CBS + 2)
    ]
    in_specs.append(pl.BlockSpec((14, 2 * C, C), lambda g: (0, 0, 0)))
    return pl.pallas_call(
        _conv_body,
        grid=(GRID // CBS,),
        in_specs=in_specs,
        out_specs=pl.BlockSpec((CROWS, C), lambda g: (g, 0)),
        out_shape=jax.ShapeDtypeStruct((ODN, C), jnp.float32),
        compiler_params=pltpu.CompilerParams(
            dimension_semantics=("parallel",)),
    )(*([dense] * (CBS + 2) + [wpair]))


def kernel(coordinates, features, weights):
    pad = NP - N
    coords_p = jnp.concatenate(
        [coordinates.astype(jnp.int32),
         jnp.full((pad, 3), GRID, jnp.int32)], axis=0)
    coords_t = coords_p.T

    dense = jax.new_ref(jnp.zeros((DN, C), jnp.float32))
    _sc_scatter(coords_t, features, dense)

    od = _conv(dense[...], weights.astype(jnp.bfloat16))

    return _sc_gather(coords_t, od)
